# Initial kernel scaffold; baseline (speedup 1.0000x reference)
import functools

import jax
import jax.numpy as jnp
import numpy as np
from jax import lax
from jax.experimental import pallas as pl
from jax.experimental.pallas import tpu as pltpu
from jax.experimental.pallas import tpu_sc as plsc

N = 10000
NE = 160000
IN_DIM = 256
OUT_DIM = 256
H = 8
DH = 32
DE = 16

DHALF = 128
ACCW = 144
NTILES = 16
EPT = NE // NTILES
C = 80
NCH = EPT // C
NPER = N // NTILES
INV_SQRT_DH = float(1.0 / np.sqrt(DH))


_BLK1 = 400


def _qkv_body(h_ref, wq_ref, wk_ref, wv_ref, q_out, k_out, v_out):
    hb = h_ref[...]
    q_out[...] = jnp.dot(hb, wq_ref[...], preferred_element_type=jnp.float32)
    k_out[...] = jnp.dot(hb, wk_ref[...], preferred_element_type=jnp.float32)
    v_out[...] = jnp.dot(hb, wv_ref[...], preferred_element_type=jnp.float32)


def _qkv_project(h, W_Q, W_K, W_V):
    nb = N // _BLK1
    grid = (nb, 2)
    in_specs = [
        pl.BlockSpec((_BLK1, IN_DIM), lambda i, j: (i, 0)),
        pl.BlockSpec((IN_DIM, DHALF), lambda i, j: (0, j)),
        pl.BlockSpec((IN_DIM, DHALF), lambda i, j: (0, j)),
        pl.BlockSpec((IN_DIM, DHALF), lambda i, j: (0, j)),
    ]
    out_spec = pl.BlockSpec((_BLK1, DHALF), lambda i, j: (j * nb + i, 0))
    out_sds = jax.ShapeDtypeStruct((2 * N, DHALF), jnp.float32)
    return pl.pallas_call(
        _qkv_body,
        grid=grid,
        in_specs=in_specs,
        out_specs=[out_spec, out_spec, out_spec],
        out_shape=[out_sds, out_sds, out_sds],
    )(h, W_Q, W_K, W_V)


_BLK2 = 2000


def _efeat_body(a_ref, we_ref, web_ref, beb_ref, e_out, b_out):
    ab = a_ref[...]
    e_out[...] = jnp.dot(ab, we_ref[...], preferred_element_type=jnp.float32)
    b_out[...] = (
        jnp.dot(ab, web_ref[...], preferred_element_type=jnp.float32)
        + beb_ref[...]
    )


def _edge_features(edge_attr, W_E, W_Eb, b_Eb):
    nb = NE // _BLK2
    grid = (nb, 2)
    in_specs = [
        pl.BlockSpec((_BLK2, DE), lambda i, j: (i, 0)),
        pl.BlockSpec((DE, DHALF), lambda i, j: (0, j)),
        pl.BlockSpec((DE, H), lambda i, j: (0, 0)),
        pl.BlockSpec((1, H), lambda i, j: (0, 0)),
    ]
    out_specs = [
        pl.BlockSpec((_BLK2, DHALF), lambda i, j: (j * nb + i, 0)),
        pl.BlockSpec((_BLK2, H), lambda i, j: (i, 0)),
    ]
    out_shape = [
        jax.ShapeDtypeStruct((2 * NE, DHALF), jnp.float32),
        jax.ShapeDtypeStruct((NE, H), jnp.float32),
    ]
    return pl.pallas_call(
        _efeat_body,
        grid=grid,
        in_specs=in_specs,
        out_specs=out_specs,
        out_shape=out_shape,
    )(edge_attr, W_E, W_Eb, b_Eb.reshape(1, H))


def _sc_body(src_h, dst_h, q_h, k_h, v_h, e_h, b_h, out_h,
             sidx, didx, didx2, kro, qro, vro, ero, bro, msg, wvacc, sem):
    cid = lax.axis_index("c")
    sid = lax.axis_index("s")
    zero16 = jnp.zeros((16,), jnp.float32)

    def _zmsg(i, carry):
        msg[i // (ACCW // 16), pl.ds((i % (ACCW // 16)) * 16, 16)] = zero16
        return carry
    lax.fori_loop(0, C * (ACCW // 16), _zmsg, 0)

    r0 = sid * NPER
    for i in range(7):
        pltpu.sync_copy(msg, wvacc.at[pl.ds(r0 + i * C, C)])
    pltpu.sync_copy(msg.at[pl.ds(0, NPER - 7 * C), :],
                    wvacc.at[pl.ds(r0 + 7 * C, NPER - 7 * C)])
    plsc.subcore_barrier()

    iota16 = lax.iota(jnp.int32, 16)
    noff = cid * N

    def _chunk(ch, carry):
        ebase = sid * EPT + ch * C
        pltpu.sync_copy(src_h.at[pl.ds(ebase, C)], sidx)
        pltpu.sync_copy(dst_h.at[pl.ds(ebase, C)], didx)

        def _adj(i, carry):
            sidx[pl.ds(i * 16, 16)] = sidx[pl.ds(i * 16, 16)] + noff
            didx2[pl.ds(i * 16, 16)] = didx[pl.ds(i * 16, 16)] + noff
            return carry
        lax.fori_loop(0, C // 16, _adj, 0)

        cpk = pltpu.async_copy(k_h.at[sidx], kro, sem)
        cpq = pltpu.async_copy(q_h.at[didx2], qro, sem)
        cpv = pltpu.async_copy(v_h.at[sidx], vro, sem)
        cpe = pltpu.async_copy(e_h.at[pl.ds(cid * NE + ebase, C)], ero, sem)
        cpb = pltpu.async_copy(b_h.at[pl.ds(ebase, C)], bro, sem)
        cpk.wait()
        cpq.wait()
        cpv.wait()
        cpe.wait()
        cpb.wait()

        for g in range(C // 16):
            rows = iota16 + (g * 16)
            for h in range(4):
                colb = h * DH

                def _dot(d8, acc):
                    for dd in range(8):
                        cvec = jnp.broadcast_to(colb + d8 * 8 + dd, (16,))
                        kk = plsc.load_gather(kro, [rows, cvec])
                        qq = plsc.load_gather(qro, [rows, cvec])
                        ee = plsc.load_gather(ero, [rows, cvec])
                        acc = acc + kk * qq * ee
                    return acc
                acc = lax.fori_loop(0, 4, _dot, zero16)

                bb = plsc.load_gather(
                    bro, [rows, jnp.broadcast_to(cid * 4 + h, (16,))])
                sc = acc * INV_SQRT_DH + bb
                sc = jnp.minimum(jnp.maximum(sc, -5.0), 5.0)
                sc = jnp.exp(sc)

                def _msg(d8, carry):
                    for dd in range(8):
                        cvec = jnp.broadcast_to(colb + d8 * 8 + dd, (16,))
                        vv = plsc.load_gather(vro, [rows, cvec])
                        plsc.store_scatter(msg, [rows, cvec], vv * sc)
                    return carry
                lax.fori_loop(0, 4, _msg, 0)
                plsc.store_scatter(
                    msg, [rows, jnp.broadcast_to(DHALF + h, (16,))], sc)

        pltpu.sync_copy(msg, wvacc.at[didx], add=True)
        return carry

    lax.fori_loop(0, NCH, _chunk, 0)

    plsc.subcore_barrier()
    pltpu.sync_copy(wvacc.at[pl.ds(r0, NPER)],
                    out_h.at[pl.ds(cid * N + r0, NPER)])


def _sc_edge_stage(src, dst, q_st, k_st, v_st, e_st, ebias):
    mesh = plsc.VectorSubcoreMesh(core_axis_name="c", subcore_axis_name="s")
    return pl.kernel(
        _sc_body,
        out_type=jax.ShapeDtypeStruct((2 * N, ACCW), jnp.float32),
        mesh=mesh,
        scratch_types=[
            pltpu.VMEM((C,), jnp.int32),
            pltpu.VMEM((C,), jnp.int32),
            pltpu.VMEM((C,), jnp.int32),
            pltpu.VMEM((C, DHALF), jnp.float32),
            pltpu.VMEM((C, DHALF), jnp.float32),
            pltpu.VMEM((C, DHALF), jnp.float32),
            pltpu.VMEM((C, DHALF), jnp.float32),
            pltpu.VMEM((C, H), jnp.float32),
            pltpu.VMEM((C, ACCW), jnp.float32),
            pltpu.VMEM_SHARED((N, ACCW), jnp.float32),
            pltpu.SemaphoreType.DMA,
        ],
    )(src, dst, q_st, k_st, v_st, e_st, ebias)


_BLK3 = 400
_BN_SCALE = float(1.0 / np.sqrt(1.0 + 1e-5))


def _final_body(h_ref, a0_ref, a1_ref, g1_ref, be1_ref, wf1_ref, bf1_ref,
                wf2_ref, bf2_ref, g2_ref, be2_ref, out_ref):
    hb = h_ref[...]
    lanes = lax.broadcasted_iota(jnp.int32, (4, DHALF), 1)
    ks = lax.broadcasted_iota(jnp.int32, (4, DHALF), 0)
    R = jnp.where(lanes // DH == ks, 1.0, 0.0).astype(jnp.float32)

    def half(a_ref):
        wv = a_ref[:, 0:DHALF]
        z = a_ref[:, DHALF:DHALF + 4]
        zx = jnp.dot(z, R, preferred_element_type=jnp.float32)
        return wv / (zx + 1e-6)

    attn = jnp.concatenate([half(a0_ref), half(a1_ref)], axis=1)
    h_attn = hb + attn
    h1 = h_attn * (g1_ref[...] * _BN_SCALE) + be1_ref[...]
    ff = jnp.maximum(
        jnp.dot(h1, wf1_ref[...], preferred_element_type=jnp.float32)
        + bf1_ref[...], 0.0)
    ff = jnp.dot(ff, wf2_ref[...], preferred_element_type=jnp.float32) \
        + bf2_ref[...]
    out_ref[...] = (h1 + ff) * (g2_ref[...] * _BN_SCALE) + be2_ref[...]


def _final_stage(h, acc, gamma1, beta1, W_ff1, b_ff1, W_ff2, b_ff2,
                 gamma2, beta2):
    nb = N // _BLK3
    grid = (nb,)
    in_specs = [
        pl.BlockSpec((_BLK3, OUT_DIM), lambda i: (i, 0)),
        pl.BlockSpec((_BLK3, ACCW), lambda i: (i, 0)),
        pl.BlockSpec((_BLK3, ACCW), lambda i: (nb + i, 0)),
        pl.BlockSpec((1, OUT_DIM), lambda i: (0, 0)),
        pl.BlockSpec((1, OUT_DIM), lambda i: (0, 0)),
        pl.BlockSpec((OUT_DIM, 2 * OUT_DIM), lambda i: (0, 0)),
        pl.BlockSpec((1, 2 * OUT_DIM), lambda i: (0, 0)),
        pl.BlockSpec((2 * OUT_DIM, OUT_DIM), lambda i: (0, 0)),
        pl.BlockSpec((1, OUT_DIM), lambda i: (0, 0)),
        pl.BlockSpec((1, OUT_DIM), lambda i: (0, 0)),
        pl.BlockSpec((1, OUT_DIM), lambda i: (0, 0)),
    ]
    return pl.pallas_call(
        _final_body,
        grid=grid,
        in_specs=in_specs,
        out_specs=pl.BlockSpec((_BLK3, OUT_DIM), lambda i: (i, 0)),
        out_shape=jax.ShapeDtypeStruct((N, OUT_DIM), jnp.float32),
    )(h, acc, acc, gamma1.reshape(1, -1), beta1.reshape(1, -1),
      W_ff1, b_ff1.reshape(1, -1), W_ff2, b_ff2.reshape(1, -1),
      gamma2.reshape(1, -1), beta2.reshape(1, -1))


def kernel(h, edge_index, edge_attr, W_Q, W_K, W_V, W_E, W_Eb, b_Eb,
           gamma1, beta1, W_ff1, b_ff1, W_ff2, b_ff2, gamma2, beta2):
    src = edge_index[0]
    dst = edge_index[1]
    q_st, k_st, v_st = _qkv_project(h, W_Q, W_K, W_V)
    e_st, ebias = _edge_features(edge_attr, W_E, W_Eb, b_Eb)
    acc = _sc_edge_stage(src, dst, q_st, k_st, v_st, e_st, ebias)
    h2 = _final_stage(h, acc, gamma1, beta1, W_ff1, b_ff1, W_ff2, b_ff2,
                      gamma2, beta2)
    return (h2, edge_attr)


# speedup vs baseline: 4.9017x; 4.9017x over previous
import functools

import jax
import jax.numpy as jnp
import numpy as np
from jax import lax
from jax.experimental import pallas as pl
from jax.experimental.pallas import tpu as pltpu
from jax.experimental.pallas import tpu_sc as plsc

N = 10000
NE = 160000
IN_DIM = 256
OUT_DIM = 256
H = 8
DH = 32
DE = 16

DHALF = 128
NTILES = 16
EPT = NE // NTILES
C = 80
NCH = EPT // C
NPER = N // NTILES
INV_SQRT_DH = float(1.0 / np.sqrt(DH))


_BLK1 = 400


def _qkv_body(h_ref, wq_ref, wk_ref, wv_ref, q_out, k_out, v_out):
    hb = h_ref[...]
    q_out[...] = jnp.dot(hb, wq_ref[...], preferred_element_type=jnp.float32)
    k_out[...] = jnp.dot(hb, wk_ref[...], preferred_element_type=jnp.float32)
    v_out[...] = jnp.dot(hb, wv_ref[...], preferred_element_type=jnp.float32)


def _qkv_project(h, W_Q, W_K, W_V):
    nb = N // _BLK1
    grid = (nb, 2)
    in_specs = [
        pl.BlockSpec((_BLK1, IN_DIM), lambda i, j: (i, 0)),
        pl.BlockSpec((IN_DIM, DHALF), lambda i, j: (0, j)),
        pl.BlockSpec((IN_DIM, DHALF), lambda i, j: (0, j)),
        pl.BlockSpec((IN_DIM, DHALF), lambda i, j: (0, j)),
    ]
    out_spec = pl.BlockSpec((_BLK1, DHALF), lambda i, j: (j * nb + i, 0))
    out_sds = jax.ShapeDtypeStruct((2 * N, DHALF), jnp.float32)
    return pl.pallas_call(
        _qkv_body,
        grid=grid,
        in_specs=in_specs,
        out_specs=[out_spec, out_spec, out_spec],
        out_shape=[out_sds, out_sds, out_sds],
    )(h, W_Q, W_K, W_V)


_BLK2 = 1280


def _efeat_body(a_ref, we_ref, webt_ref, bebt_ref, e_out, b_out):
    ab = a_ref[...]
    e_out[...] = jnp.dot(ab, we_ref[...], preferred_element_type=jnp.float32)
    bt = jax.lax.dot_general(
        webt_ref[...], ab, (((1,), (1,)), ((), ())),
        preferred_element_type=jnp.float32)
    b_out[...] = bt + bebt_ref[...]


def _edge_features(edge_attr, W_E, W_Eb, b_Eb):
    nb = NE // _BLK2
    grid = (nb, 2)
    in_specs = [
        pl.BlockSpec((_BLK2, DE), lambda i, j: (i, 0)),
        pl.BlockSpec((DE, DHALF), lambda i, j: (0, j)),
        pl.BlockSpec((H, DE), lambda i, j: (0, 0)),
        pl.BlockSpec((H, 1), lambda i, j: (0, 0)),
    ]
    out_specs = [
        pl.BlockSpec((_BLK2, DHALF), lambda i, j: (j * nb + i, 0)),
        pl.BlockSpec((H, _BLK2), lambda i, j: (0, i)),
    ]
    out_shape = [
        jax.ShapeDtypeStruct((2 * NE, DHALF), jnp.float32),
        jax.ShapeDtypeStruct((H, NE), jnp.float32),
    ]
    return pl.pallas_call(
        _efeat_body,
        grid=grid,
        in_specs=in_specs,
        out_specs=out_specs,
        out_shape=out_shape,
    )(edge_attr, W_E, W_Eb.T, b_Eb.reshape(H, 1))


def _sc_body(src_h, dst_h, q_h, k_h, v_h, e_h, b_h, wv_out, z_out,
             sidx, didx, didx2, kro, qro, ero, brot, sbuf, wvacc, zacc, sem):
    cid = lax.axis_index("c")
    sid = lax.axis_index("s")
    zero16 = jnp.zeros((16,), jnp.float32)
    iota16 = lax.iota(jnp.int32, 16)

    def _zero_e(i, carry):
        ero[i // 8, pl.ds((i % 8) * 16, 16)] = zero16
        return carry
    lax.fori_loop(0, C * 8, _zero_e, 0)

    def _zero_s(i, carry):
        sbuf[i, pl.ds(0, 16)] = zero16
        return carry
    lax.fori_loop(0, C, _zero_s, 0)

    r0 = sid * NPER
    for i in range(7):
        pltpu.sync_copy(ero, wvacc.at[pl.ds(r0 + i * C, C)])
        pltpu.sync_copy(sbuf, zacc.at[pl.ds(r0 + i * C, C)])
    rem = NPER - 7 * C
    pltpu.sync_copy(ero.at[pl.ds(0, rem), :], wvacc.at[pl.ds(r0 + 7 * C, rem)])
    pltpu.sync_copy(sbuf.at[pl.ds(0, rem), :], zacc.at[pl.ds(r0 + 7 * C, rem)])
    plsc.subcore_barrier()

    noff = cid * N

    def _chunk(ch, carry):
        ebase = sid * EPT + ch * C
        pltpu.sync_copy(src_h.at[pl.ds(ebase, C)], sidx)
        pltpu.sync_copy(dst_h.at[pl.ds(ebase, C)], didx)

        def _adj(i, carry):
            sidx[pl.ds(i * 16, 16)] = sidx[pl.ds(i * 16, 16)] + noff
            didx2[pl.ds(i * 16, 16)] = didx[pl.ds(i * 16, 16)] + noff
            return carry
        lax.fori_loop(0, C // 16, _adj, 0)

        cpk = pltpu.async_copy(k_h.at[sidx], kro, sem)
        cpq = pltpu.async_copy(q_h.at[didx2], qro, sem)
        cpe = pltpu.async_copy(e_h.at[pl.ds(cid * NE + ebase, C)], ero, sem)
        cpb = pltpu.async_copy(
            b_h.at[pl.ds(cid * 4, 4), pl.ds(ebase, C)], brot, sem)
        cpk.wait()
        cpq.wait()
        cpe.wait()
        cpb.wait()

        for g in range(C // 16):
            rows = iota16 + (g * 16)
            for h in range(4):
                colb = h * DH

                def _dot(d8, acc):
                    for dd in range(8):
                        cvec = jnp.broadcast_to(colb + d8 * 8 + dd, (16,))
                        kk = plsc.load_gather(kro, [rows, cvec])
                        qq = plsc.load_gather(qro, [rows, cvec])
                        ee = plsc.load_gather(ero, [rows, cvec])
                        acc = acc + kk * qq * ee
                    return acc
                acc = lax.fori_loop(0, 4, _dot, zero16)

                bb = brot[h, pl.ds(g * 16, 16)]
                sc = acc * INV_SQRT_DH + bb
                sc = jnp.minimum(jnp.maximum(sc, -5.0), 5.0)
                sc = jnp.exp(sc)
                plsc.store_scatter(
                    sbuf, [rows, jnp.broadcast_to(h, (16,))], sc)

        cpv = pltpu.async_copy(v_h.at[sidx], kro, sem)
        cpv.wait()
        for g in range(C // 16):
            rows = iota16 + (g * 16)
            for h in range(4):
                colb = h * DH
                sc = plsc.load_gather(
                    sbuf, [rows, jnp.broadcast_to(h, (16,))])

                def _msg(d8, carry):
                    for dd in range(8):
                        cvec = jnp.broadcast_to(colb + d8 * 8 + dd, (16,))
                        vv = plsc.load_gather(kro, [rows, cvec])
                        plsc.store_scatter(ero, [rows, cvec], vv * sc)
                    return carry
                lax.fori_loop(0, 4, _msg, 0)

        pltpu.sync_copy(ero, wvacc.at[didx], add=True)
        pltpu.sync_copy(sbuf, zacc.at[didx], add=True)
        return carry

    lax.fori_loop(0, NCH, _chunk, 0)

    plsc.subcore_barrier()
    pltpu.sync_copy(wvacc.at[pl.ds(r0, NPER)],
                    wv_out.at[pl.ds(cid * N + r0, NPER)])
    pltpu.sync_copy(zacc.at[pl.ds(r0, NPER)],
                    z_out.at[pl.ds(cid * N + r0, NPER)])


def _sc_edge_stage(src, dst, q_st, k_st, v_st, e_st, ebias_t):
    mesh = plsc.VectorSubcoreMesh(core_axis_name="c", subcore_axis_name="s",
                                  num_cores=2, num_subcores=NTILES)
    return pl.kernel(
        _sc_body,
        out_type=(jax.ShapeDtypeStruct((2 * N, DHALF), jnp.float32),
                  jax.ShapeDtypeStruct((2 * N, 16), jnp.float32)),
        mesh=mesh,
        compiler_params=pltpu.CompilerParams(use_tc_tiling_on_sc=False,
                                             needs_layout_passes=False),
        scratch_types=[
            pltpu.VMEM((C,), jnp.int32),
            pltpu.VMEM((C,), jnp.int32),
            pltpu.VMEM((C,), jnp.int32),
            pltpu.VMEM((C, DHALF), jnp.float32),
            pltpu.VMEM((C, DHALF), jnp.float32),
            pltpu.VMEM((C, DHALF), jnp.float32),
            pltpu.VMEM((4, C), jnp.float32),
            pltpu.VMEM((C, 16), jnp.float32),
            pltpu.VMEM_SHARED((N, DHALF), jnp.float32),
            pltpu.VMEM_SHARED((N, 16), jnp.float32),
            pltpu.SemaphoreType.DMA,
        ],
    )(src, dst, q_st, k_st, v_st, e_st, ebias_t)


_BLK3 = 400
_BN_SCALE = float(1.0 / np.sqrt(1.0 + 1e-5))


def _final_body(h_ref, w0_ref, w1_ref, z0_ref, z1_ref, g1_ref, be1_ref,
                wf1_ref, bf1_ref, wf2_ref, bf2_ref, g2_ref, be2_ref, out_ref):
    hb = h_ref[...]
    lanes = lax.broadcasted_iota(jnp.int32, (4, DHALF), 1)
    ks = lax.broadcasted_iota(jnp.int32, (4, DHALF), 0)
    R = jnp.where(lanes // DH == ks, 1.0, 0.0).astype(jnp.float32)

    def half(w_ref, z_ref):
        wv = w_ref[...]
        z = z_ref[:, 0:4]
        zx = jnp.dot(z, R, preferred_element_type=jnp.float32)
        return wv / (zx + 1e-6)

    attn = jnp.concatenate(
        [half(w0_ref, z0_ref), half(w1_ref, z1_ref)], axis=1)
    h_attn = hb + attn
    h1 = h_attn * (g1_ref[...] * _BN_SCALE) + be1_ref[...]
    ff = jnp.maximum(
        jnp.dot(h1, wf1_ref[...], preferred_element_type=jnp.float32)
        + bf1_ref[...], 0.0)
    ff = jnp.dot(ff, wf2_ref[...], preferred_element_type=jnp.float32) \
        + bf2_ref[...]
    out_ref[...] = (h1 + ff) * (g2_ref[...] * _BN_SCALE) + be2_ref[...]


def _final_stage(h, wv, z, gamma1, beta1, W_ff1, b_ff1, W_ff2, b_ff2,
                 gamma2, beta2):
    nb = N // _BLK3
    grid = (nb,)
    in_specs = [
        pl.BlockSpec((_BLK3, OUT_DIM), lambda i: (i, 0)),
        pl.BlockSpec((_BLK3, DHALF), lambda i: (i, 0)),
        pl.BlockSpec((_BLK3, DHALF), lambda i: (nb + i, 0)),
        pl.BlockSpec((_BLK3, 16), lambda i: (i, 0)),
        pl.BlockSpec((_BLK3, 16), lambda i: (nb + i, 0)),
        pl.BlockSpec((1, OUT_DIM), lambda i: (0, 0)),
        pl.BlockSpec((1, OUT_DIM), lambda i: (0, 0)),
        pl.BlockSpec((OUT_DIM, 2 * OUT_DIM), lambda i: (0, 0)),
        pl.BlockSpec((1, 2 * OUT_DIM), lambda i: (0, 0)),
        pl.BlockSpec((2 * OUT_DIM, OUT_DIM), lambda i: (0, 0)),
        pl.BlockSpec((1, OUT_DIM), lambda i: (0, 0)),
        pl.BlockSpec((1, OUT_DIM), lambda i: (0, 0)),
        pl.BlockSpec((1, OUT_DIM), lambda i: (0, 0)),
    ]
    return pl.pallas_call(
        _final_body,
        grid=grid,
        in_specs=in_specs,
        out_specs=pl.BlockSpec((_BLK3, OUT_DIM), lambda i: (i, 0)),
        out_shape=jax.ShapeDtypeStruct((N, OUT_DIM), jnp.float32),
    )(h, wv, wv, z, z, gamma1.reshape(1, -1), beta1.reshape(1, -1),
      W_ff1, b_ff1.reshape(1, -1), W_ff2, b_ff2.reshape(1, -1),
      gamma2.reshape(1, -1), beta2.reshape(1, -1))


def kernel(h, edge_index, edge_attr, W_Q, W_K, W_V, W_E, W_Eb, b_Eb,
           gamma1, beta1, W_ff1, b_ff1, W_ff2, b_ff2, gamma2, beta2):
    src = edge_index[0]
    dst = edge_index[1]
    q_st, k_st, v_st = _qkv_project(h, W_Q, W_K, W_V)
    e_st, ebias_t = _edge_features(edge_attr, W_E, W_Eb, b_Eb)
    wv, z = _sc_edge_stage(src, dst, q_st, k_st, v_st, e_st, ebias_t)
    h2 = _final_stage(h, wv, z, gamma1, beta1, W_ff1, b_ff1, W_ff2, b_ff2,
                      gamma2, beta2)
    return (h2, edge_attr)


# speedup vs baseline: 16.3162x; 3.3287x over previous
import functools

import jax
import jax.numpy as jnp
import numpy as np
from jax import lax
from jax.experimental import pallas as pl
from jax.experimental.pallas import tpu as pltpu
from jax.experimental.pallas import tpu_sc as plsc

N = 10000
NE = 160000
IN_DIM = 256
OUT_DIM = 256
H = 8
DH = 32
DE = 16

DHALF = 128
NTILES = 16
C = 64
EPT0 = 9984
NCH0 = EPT0 // C
NPER = N // NTILES
INV_SQRT_DH = float(1.0 / np.sqrt(DH))


_BLK1 = 400


def _qkv_body(h_ref, wq_ref, wk_ref, wv_ref, q_out, k_out, v_out):
    hb = h_ref[...]
    q_out[...] = jnp.dot(hb, wq_ref[...], preferred_element_type=jnp.float32)
    k_out[...] = jnp.dot(hb, wk_ref[...], preferred_element_type=jnp.float32)
    v_out[...] = jnp.dot(hb, wv_ref[...], preferred_element_type=jnp.float32)


def _qkv_project(h, W_Q, W_K, W_V):
    nb = N // _BLK1
    grid = (nb, 2)
    in_specs = [
        pl.BlockSpec((_BLK1, IN_DIM), lambda i, j: (i, 0)),
        pl.BlockSpec((IN_DIM, DHALF), lambda i, j: (0, j)),
        pl.BlockSpec((IN_DIM, DHALF), lambda i, j: (0, j)),
        pl.BlockSpec((IN_DIM, DHALF), lambda i, j: (0, j)),
    ]
    out_spec = pl.BlockSpec((_BLK1, DHALF), lambda i, j: (j * nb + i, 0))
    out_sds = jax.ShapeDtypeStruct((2 * N, DHALF), jnp.float32)
    return pl.pallas_call(
        _qkv_body,
        grid=grid,
        in_specs=in_specs,
        out_specs=[out_spec, out_spec, out_spec],
        out_shape=[out_sds, out_sds, out_sds],
    )(h, W_Q, W_K, W_V)


_BLK2 = 1280


def _efeat_body(a_ref, we_ref, webt_ref, bebt_ref, e_out, b_out):
    ab = a_ref[...]
    e_out[...] = jnp.dot(ab, we_ref[...], preferred_element_type=jnp.float32)
    bt = jax.lax.dot_general(
        webt_ref[...], ab, (((1,), (1,)), ((), ())),
        preferred_element_type=jnp.float32)
    b_out[...] = bt + bebt_ref[...]


def _edge_features(edge_attr, W_E, W_Eb, b_Eb):
    nb = NE // _BLK2
    grid = (nb, 2)
    in_specs = [
        pl.BlockSpec((_BLK2, DE), lambda i, j: (i, 0)),
        pl.BlockSpec((DE, DHALF), lambda i, j: (0, j)),
        pl.BlockSpec((H, DE), lambda i, j: (0, 0)),
        pl.BlockSpec((H, 1), lambda i, j: (0, 0)),
    ]
    out_specs = [
        pl.BlockSpec((_BLK2, DHALF), lambda i, j: (j * nb + i, 0)),
        pl.BlockSpec((H, _BLK2), lambda i, j: (0, i)),
    ]
    out_shape = [
        jax.ShapeDtypeStruct((2 * NE, DHALF), jnp.float32),
        jax.ShapeDtypeStruct((H, NE), jnp.float32),
    ]
    return pl.pallas_call(
        _efeat_body,
        grid=grid,
        in_specs=in_specs,
        out_specs=out_specs,
        out_shape=out_shape,
    )(edge_attr, W_E, W_Eb.T, b_Eb.reshape(H, 1))


def _sc_body(src_h, dst_h, q_h, k_h, v_h, e_h, b_h, wv_out, z_out,
             sidx, didx, didx2, kro, qro, ero, msg, brot, sbuf, sbuft,
             wvacc, zacc, sem):
    cid = lax.axis_index("c")
    sid = lax.axis_index("s")
    zero16 = jnp.zeros((16,), jnp.float32)
    iota16 = lax.iota(jnp.int32, 16)

    def _zero_m(i, carry):
        msg[i // 8, pl.ds((i % 8) * 16, 16)] = zero16
        return carry
    lax.fori_loop(0, C * 8, _zero_m, 0)

    def _zero_s(i, carry):
        sbuf[i, pl.ds(0, 16)] = zero16
        return carry
    lax.fori_loop(0, C, _zero_s, 0)

    r0 = sid * NPER
    for i in range(9):
        pltpu.sync_copy(msg, wvacc.at[pl.ds(r0 + i * C, C)])
        pltpu.sync_copy(sbuf, zacc.at[pl.ds(r0 + i * C, C)])
    rem = NPER - 9 * C
    pltpu.sync_copy(msg.at[pl.ds(0, rem), :], wvacc.at[pl.ds(r0 + 9 * C, rem)])
    pltpu.sync_copy(sbuf.at[pl.ds(0, rem), :], zacc.at[pl.ds(r0 + 9 * C, rem)])
    plsc.subcore_barrier()

    noff = cid * N

    def _chunk(ch, carry):
        ebase = sid * EPT0 + ch * C
        cpsi = pltpu.async_copy(src_h.at[pl.ds(ebase, C)], sidx, sem)
        cpdi = pltpu.async_copy(dst_h.at[pl.ds(ebase, C)], didx, sem)
        cpsi.wait()
        cpdi.wait()

        def _adj(i, carry):
            sidx[pl.ds(i * 16, 16)] = sidx[pl.ds(i * 16, 16)] + noff
            didx2[pl.ds(i * 16, 16)] = didx[pl.ds(i * 16, 16)] + noff
            return carry
        lax.fori_loop(0, C // 16, _adj, 0)

        cpk = pltpu.async_copy(k_h.at[sidx], kro, sem)
        cpq = pltpu.async_copy(q_h.at[didx2], qro, sem)
        cpe = pltpu.async_copy(e_h.at[pl.ds(cid * NE + ebase, C)], ero, sem)
        cpb = pltpu.async_copy(
            b_h.at[pl.ds(cid * 4, 4), pl.ds(ebase, C)], brot, sem)
        cpk.wait()
        cpq.wait()
        cpe.wait()
        cpb.wait()

        lane0 = iota16 == 0

        def _p1(e, carry):
            evec = jnp.broadcast_to(e, (16,))
            p = []
            for sl in range(8):
                kk = kro[e, pl.ds(sl * 16, 16)]
                qq = qro[e, pl.ds(sl * 16, 16)]
                ee = ero[e, pl.ds(sl * 16, 16)]
                p.append(kk * qq * ee)
            for h in range(4):
                s16 = p[2 * h] + p[2 * h + 1]
                r = jnp.broadcast_to(jnp.sum(s16), (16,))
                plsc.store_scatter(
                    sbuft, [jnp.broadcast_to(h, (16,)), evec], r, mask=lane0)
            return carry
        lax.fori_loop(0, C, _p1, 0)

        cpv = pltpu.async_copy(v_h.at[sidx], kro, sem)

        for h in range(4):
            for sl in range(C // 16):
                x = sbuft[h, pl.ds(sl * 16, 16)]
                bb = brot[h, pl.ds(sl * 16, 16)]
                x = x * INV_SQRT_DH + bb
                x = jnp.minimum(jnp.maximum(x, -5.0), 5.0)
                x = jnp.exp(x)
                sbuft[h, pl.ds(sl * 16, 16)] = x
                plsc.store_scatter(
                    sbuf, [iota16 + sl * 16, jnp.broadcast_to(h, (16,))], x)

        cpv.wait()

        def _p2(g, carry):
            rows0 = g * 16
            srows = [sbuft[h, pl.ds(rows0, 16)] for h in range(4)]
            for el in range(16):
                e = rows0 + el
                for h in range(4):
                    s = jnp.broadcast_to(srows[h][el], (16,))
                    msg[e, pl.ds(h * 32, 16)] = kro[e, pl.ds(h * 32, 16)] * s
                    msg[e, pl.ds(h * 32 + 16, 16)] = \
                        kro[e, pl.ds(h * 32 + 16, 16)] * s
            return carry
        lax.fori_loop(0, C // 16, _p2, 0)

        pltpu.sync_copy(msg, wvacc.at[didx], add=True)
        pltpu.sync_copy(sbuf, zacc.at[didx], add=True)
        return carry

    nch = NCH0 + jnp.where(sid == NTILES - 1, 4, 0)
    lax.fori_loop(0, nch, _chunk, 0)

    plsc.subcore_barrier()
    pltpu.sync_copy(wvacc.at[pl.ds(r0, NPER)],
                    wv_out.at[pl.ds(cid * N + r0, NPER)])
    pltpu.sync_copy(zacc.at[pl.ds(r0, NPER)],
                    z_out.at[pl.ds(cid * N + r0, NPER)])


def _sc_edge_stage(src, dst, q_st, k_st, v_st, e_st, ebias_t):
    mesh = plsc.VectorSubcoreMesh(core_axis_name="c", subcore_axis_name="s",
                                  num_cores=2, num_subcores=NTILES)
    return pl.kernel(
        _sc_body,
        out_type=(jax.ShapeDtypeStruct((2 * N, DHALF), jnp.float32),
                  jax.ShapeDtypeStruct((2 * N, 16), jnp.float32)),
        mesh=mesh,
        compiler_params=pltpu.CompilerParams(use_tc_tiling_on_sc=False,
                                             needs_layout_passes=False),
        scratch_types=[
            pltpu.VMEM((C,), jnp.int32),
            pltpu.VMEM((C,), jnp.int32),
            pltpu.VMEM((C,), jnp.int32),
            pltpu.VMEM((C, DHALF), jnp.float32),
            pltpu.VMEM((C, DHALF), jnp.float32),
            pltpu.VMEM((C, DHALF), jnp.float32),
            pltpu.VMEM((C, DHALF), jnp.float32),
            pltpu.VMEM((4, C), jnp.float32),
            pltpu.VMEM((C, 16), jnp.float32),
            pltpu.VMEM((4, C), jnp.float32),
            pltpu.VMEM_SHARED((N, DHALF), jnp.float32),
            pltpu.VMEM_SHARED((N, 16), jnp.float32),
            pltpu.SemaphoreType.DMA,
        ],
    )(src, dst, q_st, k_st, v_st, e_st, ebias_t)


_BLK3 = 400
_BN_SCALE = float(1.0 / np.sqrt(1.0 + 1e-5))


def _final_body(h_ref, w0_ref, w1_ref, z0_ref, z1_ref, g1_ref, be1_ref,
                wf1_ref, bf1_ref, wf2_ref, bf2_ref, g2_ref, be2_ref, out_ref):
    hb = h_ref[...]
    lanes = lax.broadcasted_iota(jnp.int32, (4, DHALF), 1)
    ks = lax.broadcasted_iota(jnp.int32, (4, DHALF), 0)
    R = jnp.where(lanes // DH == ks, 1.0, 0.0).astype(jnp.float32)

    def half(w_ref, z_ref):
        wv = w_ref[...]
        z = z_ref[:, 0:4]
        zx = jnp.dot(z, R, preferred_element_type=jnp.float32)
        return wv / (zx + 1e-6)

    attn = jnp.concatenate(
        [half(w0_ref, z0_ref), half(w1_ref, z1_ref)], axis=1)
    h_attn = hb + attn
    h1 = h_attn * (g1_ref[...] * _BN_SCALE) + be1_ref[...]
    ff = jnp.maximum(
        jnp.dot(h1, wf1_ref[...], preferred_element_type=jnp.float32)
        + bf1_ref[...], 0.0)
    ff = jnp.dot(ff, wf2_ref[...], preferred_element_type=jnp.float32) \
        + bf2_ref[...]
    out_ref[...] = (h1 + ff) * (g2_ref[...] * _BN_SCALE) + be2_ref[...]


def _final_stage(h, wv, z, gamma1, beta1, W_ff1, b_ff1, W_ff2, b_ff2,
                 gamma2, beta2):
    nb = N // _BLK3
    grid = (nb,)
    in_specs = [
        pl.BlockSpec((_BLK3, OUT_DIM), lambda i: (i, 0)),
        pl.BlockSpec((_BLK3, DHALF), lambda i: (i, 0)),
        pl.BlockSpec((_BLK3, DHALF), lambda i: (nb + i, 0)),
        pl.BlockSpec((_BLK3, 16), lambda i: (i, 0)),
        pl.BlockSpec((_BLK3, 16), lambda i: (nb + i, 0)),
        pl.BlockSpec((1, OUT_DIM), lambda i: (0, 0)),
        pl.BlockSpec((1, OUT_DIM), lambda i: (0, 0)),
        pl.BlockSpec((OUT_DIM, 2 * OUT_DIM), lambda i: (0, 0)),
        pl.BlockSpec((1, 2 * OUT_DIM), lambda i: (0, 0)),
        pl.BlockSpec((2 * OUT_DIM, OUT_DIM), lambda i: (0, 0)),
        pl.BlockSpec((1, OUT_DIM), lambda i: (0, 0)),
        pl.BlockSpec((1, OUT_DIM), lambda i: (0, 0)),
        pl.BlockSpec((1, OUT_DIM), lambda i: (0, 0)),
    ]
    return pl.pallas_call(
        _final_body,
        grid=grid,
        in_specs=in_specs,
        out_specs=pl.BlockSpec((_BLK3, OUT_DIM), lambda i: (i, 0)),
        out_shape=jax.ShapeDtypeStruct((N, OUT_DIM), jnp.float32),
    )(h, wv, wv, z, z, gamma1.reshape(1, -1), beta1.reshape(1, -1),
      W_ff1, b_ff1.reshape(1, -1), W_ff2, b_ff2.reshape(1, -1),
      gamma2.reshape(1, -1), beta2.reshape(1, -1))


def kernel(h, edge_index, edge_attr, W_Q, W_K, W_V, W_E, W_Eb, b_Eb,
           gamma1, beta1, W_ff1, b_ff1, W_ff2, b_ff2, gamma2, beta2):
    src = edge_index[0]
    dst = edge_index[1]
    q_st, k_st, v_st = _qkv_project(h, W_Q, W_K, W_V)
    e_st, ebias_t = _edge_features(edge_attr, W_E, W_Eb, b_Eb)
    wv, z = _sc_edge_stage(src, dst, q_st, k_st, v_st, e_st, ebias_t)
    h2 = _final_stage(h, wv, z, gamma1, beta1, W_ff1, b_ff1, W_ff2, b_ff2,
                      gamma2, beta2)
    return (h2, edge_attr)


# speedup vs baseline: 17.1020x; 1.0482x over previous
import functools

import jax
import jax.numpy as jnp
import numpy as np
from jax import lax
from jax.experimental import pallas as pl
from jax.experimental.pallas import tpu as pltpu
from jax.experimental.pallas import tpu_sc as plsc

N = 10000
NE = 160000
IN_DIM = 256
OUT_DIM = 256
H = 8
DH = 32
DE = 16

DHALF = 128
NTILES = 16
C = 64
EPT0 = 9984
NCH0 = EPT0 // C
NPER = N // NTILES
INV_SQRT_DH = float(1.0 / np.sqrt(DH))


_BLK1 = 400


def _qkv_body(h_ref, wq_ref, wk_ref, wv_ref, q_out, k_out, v_out):
    hb = h_ref[...]
    q_out[...] = jnp.dot(hb, wq_ref[...], preferred_element_type=jnp.float32)
    k_out[...] = jnp.dot(hb, wk_ref[...], preferred_element_type=jnp.float32)
    v_out[...] = jnp.dot(hb, wv_ref[...], preferred_element_type=jnp.float32)


def _qkv_project(h, W_Q, W_K, W_V):
    nb = N // _BLK1
    grid = (nb, 2)
    in_specs = [
        pl.BlockSpec((_BLK1, IN_DIM), lambda i, j: (i, 0)),
        pl.BlockSpec((IN_DIM, DHALF), lambda i, j: (0, j)),
        pl.BlockSpec((IN_DIM, DHALF), lambda i, j: (0, j)),
        pl.BlockSpec((IN_DIM, DHALF), lambda i, j: (0, j)),
    ]
    out_spec = pl.BlockSpec((_BLK1, DHALF), lambda i, j: (j * nb + i, 0))
    out_sds = jax.ShapeDtypeStruct((2 * N, DHALF), jnp.float32)
    return pl.pallas_call(
        _qkv_body,
        grid=grid,
        in_specs=in_specs,
        out_specs=[out_spec, out_spec, out_spec],
        out_shape=[out_sds, out_sds, out_sds],
    )(h, W_Q, W_K, W_V)


_BLK2 = 1280


def _efeat_body(a_ref, we_ref, webt_ref, bebt_ref, e_out, b_out):
    ab = a_ref[...]
    e_out[...] = jnp.dot(ab, we_ref[...], preferred_element_type=jnp.float32)
    bt = jax.lax.dot_general(
        webt_ref[...], ab, (((1,), (1,)), ((), ())),
        preferred_element_type=jnp.float32)
    b_out[...] = bt + bebt_ref[...]


def _edge_features(edge_attr, W_E, W_Eb, b_Eb):
    nb = NE // _BLK2
    grid = (nb, 2)
    in_specs = [
        pl.BlockSpec((_BLK2, DE), lambda i, j: (i, 0)),
        pl.BlockSpec((DE, DHALF), lambda i, j: (0, j)),
        pl.BlockSpec((H, DE), lambda i, j: (0, 0)),
        pl.BlockSpec((H, 1), lambda i, j: (0, 0)),
    ]
    out_specs = [
        pl.BlockSpec((_BLK2, DHALF), lambda i, j: (j * nb + i, 0)),
        pl.BlockSpec((H, _BLK2), lambda i, j: (0, i)),
    ]
    out_shape = [
        jax.ShapeDtypeStruct((2 * NE, DHALF), jnp.float32),
        jax.ShapeDtypeStruct((H, NE), jnp.float32),
    ]
    return pl.pallas_call(
        _efeat_body,
        grid=grid,
        in_specs=in_specs,
        out_specs=out_specs,
        out_shape=out_shape,
    )(edge_attr, W_E, W_Eb.T, b_Eb.reshape(H, 1))


def _sc_body(src_h, dst_h, q_h, k_h, v_h, e_h, b_h, wv_out, z_out,
             sidx, didxa, didxb, didx2, kro, qro, ero, msg, brot, sbuf, sbuft,
             wvacc, zacc, semg, semv, semw, semz):
    cid = lax.axis_index("c")
    sid = lax.axis_index("s")
    zero16 = jnp.zeros((16,), jnp.float32)
    iota16 = lax.iota(jnp.int32, 16)

    def _zero_m(i, carry):
        msg[i // 8, pl.ds((i % 8) * 16, 16)] = zero16
        return carry
    lax.fori_loop(0, C * 8, _zero_m, 0)

    def _zero_s(i, carry):
        sbuf[i, pl.ds(0, 16)] = zero16
        return carry
    lax.fori_loop(0, C, _zero_s, 0)

    r0 = sid * NPER
    for i in range(9):
        pltpu.sync_copy(msg, wvacc.at[pl.ds(r0 + i * C, C)])
        pltpu.sync_copy(sbuf, zacc.at[pl.ds(r0 + i * C, C)])
    rem = NPER - 9 * C
    pltpu.sync_copy(msg.at[pl.ds(0, rem), :], wvacc.at[pl.ds(r0 + 9 * C, rem)])
    pltpu.sync_copy(sbuf.at[pl.ds(0, rem), :], zacc.at[pl.ds(r0 + 9 * C, rem)])
    plsc.subcore_barrier()

    noff = cid * N
    lane0 = iota16 == 0

    def _do_chunk(ch, didx, prev_didx):
        ebase = sid * EPT0 + ch * C
        cpsi = pltpu.async_copy(src_h.at[pl.ds(ebase, C)], sidx, semg)
        cpdi = pltpu.async_copy(dst_h.at[pl.ds(ebase, C)], didx, semg)
        cpsi.wait()
        cpdi.wait()

        def _adj(i, carry):
            sidx[pl.ds(i * 16, 16)] = sidx[pl.ds(i * 16, 16)] + noff
            didx2[pl.ds(i * 16, 16)] = didx[pl.ds(i * 16, 16)] + noff
            return carry
        lax.fori_loop(0, C // 16, _adj, 0)

        cpk = pltpu.async_copy(k_h.at[sidx], kro, semg)
        cpq = pltpu.async_copy(q_h.at[didx2], qro, semg)
        cpe = pltpu.async_copy(e_h.at[pl.ds(cid * NE + ebase, C)], ero, semg)
        cpb = pltpu.async_copy(
            b_h.at[pl.ds(cid * 4, 4), pl.ds(ebase, C)], brot, semg)
        cpk.wait()
        cpq.wait()
        cpe.wait()
        cpb.wait()

        def _p1(e, carry):
            evec = jnp.broadcast_to(e, (16,))
            p = []
            for sl in range(8):
                kk = kro[e, pl.ds(sl * 16, 16)]
                qq = qro[e, pl.ds(sl * 16, 16)]
                ee = ero[e, pl.ds(sl * 16, 16)]
                p.append(kk * qq * ee)
            for h in range(4):
                s16 = p[2 * h] + p[2 * h + 1]
                r = jnp.broadcast_to(jnp.sum(s16), (16,))
                plsc.store_scatter(
                    sbuft, [jnp.broadcast_to(h, (16,)), evec], r, mask=lane0)
            return carry
        lax.fori_loop(0, C, _p1, 0)

        cpv = pltpu.async_copy(v_h.at[sidx], kro, semv)

        @pl.when(ch > 0)
        def _():
            pltpu.make_async_copy(sbuf, zacc.at[prev_didx], semz).wait()

        for h in range(4):
            for sl in range(C // 16):
                x = sbuft[h, pl.ds(sl * 16, 16)]
                bb = brot[h, pl.ds(sl * 16, 16)]
                x = x * INV_SQRT_DH + bb
                x = jnp.minimum(jnp.maximum(x, -5.0), 5.0)
                x = jnp.exp(x)
                sbuft[h, pl.ds(sl * 16, 16)] = x
                plsc.store_scatter(
                    sbuf, [iota16 + sl * 16, jnp.broadcast_to(h, (16,))], x)

        @pl.when(ch > 0)
        def _():
            pltpu.make_async_copy(msg, wvacc.at[prev_didx], semw).wait()

        cpv.wait()

        def _p2(g, carry):
            rows0 = g * 16
            srows = [sbuft[h, pl.ds(rows0, 16)] for h in range(4)]
            for el in range(16):
                e = rows0 + el
                for h in range(4):
                    s = jnp.broadcast_to(srows[h][el], (16,))
                    msg[e, pl.ds(h * 32, 16)] = kro[e, pl.ds(h * 32, 16)] * s
                    msg[e, pl.ds(h * 32 + 16, 16)] = \
                        kro[e, pl.ds(h * 32 + 16, 16)] * s
            return carry
        lax.fori_loop(0, C // 16, _p2, 0)

        pltpu.async_copy(msg, wvacc.at[didx], semw, add=True)
        pltpu.async_copy(sbuf, zacc.at[didx], semz, add=True)

    def _pair(j, carry):
        _do_chunk(2 * j, didxa, didxb)
        _do_chunk(2 * j + 1, didxb, didxa)
        return carry

    npair = (NCH0 // 2) + jnp.where(sid == NTILES - 1, 2, 0)
    lax.fori_loop(0, npair, _pair, 0)

    pltpu.make_async_copy(msg, wvacc.at[didxb], semw).wait()
    pltpu.make_async_copy(sbuf, zacc.at[didxb], semz).wait()

    plsc.subcore_barrier()
    pltpu.sync_copy(wvacc.at[pl.ds(r0, NPER)],
                    wv_out.at[pl.ds(cid * N + r0, NPER)])
    pltpu.sync_copy(zacc.at[pl.ds(r0, NPER)],
                    z_out.at[pl.ds(cid * N + r0, NPER)])


def _sc_edge_stage(src, dst, q_st, k_st, v_st, e_st, ebias_t):
    mesh = plsc.VectorSubcoreMesh(core_axis_name="c", subcore_axis_name="s",
                                  num_cores=2, num_subcores=NTILES)
    return pl.kernel(
        _sc_body,
        out_type=(jax.ShapeDtypeStruct((2 * N, DHALF), jnp.float32),
                  jax.ShapeDtypeStruct((2 * N, 16), jnp.float32)),
        mesh=mesh,
        compiler_params=pltpu.CompilerParams(use_tc_tiling_on_sc=False,
                                             needs_layout_passes=False),
        scratch_types=[
            pltpu.VMEM((C,), jnp.int32),
            pltpu.VMEM((C,), jnp.int32),
            pltpu.VMEM((C,), jnp.int32),
            pltpu.VMEM((C,), jnp.int32),
            pltpu.VMEM((C, DHALF), jnp.float32),
            pltpu.VMEM((C, DHALF), jnp.float32),
            pltpu.VMEM((C, DHALF), jnp.float32),
            pltpu.VMEM((C, DHALF), jnp.float32),
            pltpu.VMEM((4, C), jnp.float32),
            pltpu.VMEM((C, 16), jnp.float32),
            pltpu.VMEM((4, C), jnp.float32),
            pltpu.VMEM_SHARED((N, DHALF), jnp.float32),
            pltpu.VMEM_SHARED((N, 16), jnp.float32),
            pltpu.SemaphoreType.DMA,
            pltpu.SemaphoreType.DMA,
            pltpu.SemaphoreType.DMA,
            pltpu.SemaphoreType.DMA,
        ],
    )(src, dst, q_st, k_st, v_st, e_st, ebias_t)


_BLK3 = 400
_BN_SCALE = float(1.0 / np.sqrt(1.0 + 1e-5))


def _final_body(h_ref, w0_ref, w1_ref, z0_ref, z1_ref, g1_ref, be1_ref,
                wf1_ref, bf1_ref, wf2_ref, bf2_ref, g2_ref, be2_ref, out_ref):
    hb = h_ref[...]
    lanes = lax.broadcasted_iota(jnp.int32, (4, DHALF), 1)
    ks = lax.broadcasted_iota(jnp.int32, (4, DHALF), 0)
    R = jnp.where(lanes // DH == ks, 1.0, 0.0).astype(jnp.float32)

    def half(w_ref, z_ref):
        wv = w_ref[...]
        z = z_ref[:, 0:4]
        zx = jnp.dot(z, R, preferred_element_type=jnp.float32)
        return wv / (zx + 1e-6)

    attn = jnp.concatenate(
        [half(w0_ref, z0_ref), half(w1_ref, z1_ref)], axis=1)
    h_attn = hb + attn
    h1 = h_attn * (g1_ref[...] * _BN_SCALE) + be1_ref[...]
    ff = jnp.maximum(
        jnp.dot(h1, wf1_ref[...], preferred_element_type=jnp.float32)
        + bf1_ref[...], 0.0)
    ff = jnp.dot(ff, wf2_ref[...], preferred_element_type=jnp.float32) \
        + bf2_ref[...]
    out_ref[...] = (h1 + ff) * (g2_ref[...] * _BN_SCALE) + be2_ref[...]


def _final_stage(h, wv, z, gamma1, beta1, W_ff1, b_ff1, W_ff2, b_ff2,
                 gamma2, beta2):
    nb = N // _BLK3
    grid = (nb,)
    in_specs = [
        pl.BlockSpec((_BLK3, OUT_DIM), lambda i: (i, 0)),
        pl.BlockSpec((_BLK3, DHALF), lambda i: (i, 0)),
        pl.BlockSpec((_BLK3, DHALF), lambda i: (nb + i, 0)),
        pl.BlockSpec((_BLK3, 16), lambda i: (i, 0)),
        pl.BlockSpec((_BLK3, 16), lambda i: (nb + i, 0)),
        pl.BlockSpec((1, OUT_DIM), lambda i: (0, 0)),
        pl.BlockSpec((1, OUT_DIM), lambda i: (0, 0)),
        pl.BlockSpec((OUT_DIM, 2 * OUT_DIM), lambda i: (0, 0)),
        pl.BlockSpec((1, 2 * OUT_DIM), lambda i: (0, 0)),
        pl.BlockSpec((2 * OUT_DIM, OUT_DIM), lambda i: (0, 0)),
        pl.BlockSpec((1, OUT_DIM), lambda i: (0, 0)),
        pl.BlockSpec((1, OUT_DIM), lambda i: (0, 0)),
        pl.BlockSpec((1, OUT_DIM), lambda i: (0, 0)),
    ]
    return pl.pallas_call(
        _final_body,
        grid=grid,
        in_specs=in_specs,
        out_specs=pl.BlockSpec((_BLK3, OUT_DIM), lambda i: (i, 0)),
        out_shape=jax.ShapeDtypeStruct((N, OUT_DIM), jnp.float32),
    )(h, wv, wv, z, z, gamma1.reshape(1, -1), beta1.reshape(1, -1),
      W_ff1, b_ff1.reshape(1, -1), W_ff2, b_ff2.reshape(1, -1),
      gamma2.reshape(1, -1), beta2.reshape(1, -1))


def kernel(h, edge_index, edge_attr, W_Q, W_K, W_V, W_E, W_Eb, b_Eb,
           gamma1, beta1, W_ff1, b_ff1, W_ff2, b_ff2, gamma2, beta2):
    src = edge_index[0]
    dst = edge_index[1]
    q_st, k_st, v_st = _qkv_project(h, W_Q, W_K, W_V)
    e_st, ebias_t = _edge_features(edge_attr, W_E, W_Eb, b_Eb)
    wv, z = _sc_edge_stage(src, dst, q_st, k_st, v_st, e_st, ebias_t)
    h2 = _final_stage(h, wv, z, gamma1, beta1, W_ff1, b_ff1, W_ff2, b_ff2,
                      gamma2, beta2)
    return (h2, edge_attr)


# speedup vs baseline: 18.9867x; 1.1102x over previous
import functools

import jax
import jax.numpy as jnp
import numpy as np
from jax import lax
from jax.experimental import pallas as pl
from jax.experimental.pallas import tpu as pltpu
from jax.experimental.pallas import tpu_sc as plsc

N = 10000
NE = 160000
IN_DIM = 256
OUT_DIM = 256
H = 8
DH = 32
DE = 16

DHALF = 128
NTILES = 16
C = 64
EPT0 = 9984
NCH0 = EPT0 // C
NPER = N // NTILES
INV_SQRT_DH = float(1.0 / np.sqrt(DH))


_BLK1 = 400


def _qkv_body(h_ref, wq_ref, wk_ref, wv_ref, q_out, k_out, v_out):
    hb = h_ref[...]
    q_out[...] = jnp.dot(hb, wq_ref[...], preferred_element_type=jnp.float32)
    k_out[...] = jnp.dot(hb, wk_ref[...], preferred_element_type=jnp.float32)
    v_out[...] = jnp.dot(hb, wv_ref[...], preferred_element_type=jnp.float32)


def _qkv_project(h, W_Q, W_K, W_V):
    nb = N // _BLK1
    grid = (nb, 2)
    in_specs = [
        pl.BlockSpec((_BLK1, IN_DIM), lambda i, j: (i, 0)),
        pl.BlockSpec((IN_DIM, DHALF), lambda i, j: (0, j)),
        pl.BlockSpec((IN_DIM, DHALF), lambda i, j: (0, j)),
        pl.BlockSpec((IN_DIM, DHALF), lambda i, j: (0, j)),
    ]
    out_spec = pl.BlockSpec((_BLK1, DHALF), lambda i, j: (j * nb + i, 0))
    out_sds = jax.ShapeDtypeStruct((2 * N, DHALF), jnp.float32)
    return pl.pallas_call(
        _qkv_body,
        grid=grid,
        in_specs=in_specs,
        out_specs=[out_spec, out_spec, out_spec],
        out_shape=[out_sds, out_sds, out_sds],
    )(h, W_Q, W_K, W_V)


_BLK2 = 1280


def _efeat_body(a_ref, we_ref, webt_ref, bebt_ref, e_out, b_out):
    ab = a_ref[...]
    e_out[...] = jnp.dot(ab, we_ref[...], preferred_element_type=jnp.float32)
    bt = jax.lax.dot_general(
        webt_ref[...], ab, (((1,), (1,)), ((), ())),
        preferred_element_type=jnp.float32)
    b_out[...] = bt + bebt_ref[...]


def _edge_features(edge_attr, W_E, W_Eb, b_Eb):
    nb = NE // _BLK2
    grid = (nb, 2)
    in_specs = [
        pl.BlockSpec((_BLK2, DE), lambda i, j: (i, 0)),
        pl.BlockSpec((DE, DHALF), lambda i, j: (0, j)),
        pl.BlockSpec((H, DE), lambda i, j: (0, 0)),
        pl.BlockSpec((H, 1), lambda i, j: (0, 0)),
    ]
    out_specs = [
        pl.BlockSpec((_BLK2, DHALF), lambda i, j: (j * nb + i, 0)),
        pl.BlockSpec((H, _BLK2), lambda i, j: (0, i)),
    ]
    out_shape = [
        jax.ShapeDtypeStruct((2 * NE, DHALF), jnp.float32),
        jax.ShapeDtypeStruct((H, NE), jnp.float32),
    ]
    return pl.pallas_call(
        _efeat_body,
        grid=grid,
        in_specs=in_specs,
        out_specs=out_specs,
        out_shape=out_shape,
    )(edge_attr, W_E, W_Eb.T, b_Eb.reshape(H, 1))


def _sc_body(src_h, dst_h, q_h, k_h, v_h, e_h, b_h, wv_out, z_out,
             sidx, didxa, didxb, didx2, kro, qro, ero, msg, brot, sbuf, sbuft,
             wvacc, zacc, semg, semv, semw, semz):
    cid = lax.axis_index("c")
    sid = lax.axis_index("s")
    zero16 = jnp.zeros((16,), jnp.float32)
    iota16 = lax.iota(jnp.int32, 16)

    def _zero_m(i, carry):
        msg[i // 8, pl.ds((i % 8) * 16, 16)] = zero16
        return carry
    lax.fori_loop(0, C * 8, _zero_m, 0)

    def _zero_s(i, carry):
        sbuf[i, pl.ds(0, 16)] = zero16
        return carry
    lax.fori_loop(0, C, _zero_s, 0)

    r0 = sid * NPER
    for i in range(9):
        pltpu.sync_copy(msg, wvacc.at[pl.ds(r0 + i * C, C)])
        pltpu.sync_copy(sbuf, zacc.at[pl.ds(r0 + i * C, C)])
    rem = NPER - 9 * C
    pltpu.sync_copy(msg.at[pl.ds(0, rem), :], wvacc.at[pl.ds(r0 + 9 * C, rem)])
    pltpu.sync_copy(sbuf.at[pl.ds(0, rem), :], zacc.at[pl.ds(r0 + 9 * C, rem)])
    plsc.subcore_barrier()

    noff = cid * N
    lane0 = iota16 == 0

    def _do_chunk(ch, didx, prev_didx):
        ebase = sid * EPT0 + ch * C
        cpsi = pltpu.async_copy(src_h.at[pl.ds(ebase, C)], sidx, semg)
        cpdi = pltpu.async_copy(dst_h.at[pl.ds(ebase, C)], didx, semg)
        cpsi.wait()
        cpdi.wait()

        def _adj(i, carry):
            sidx[pl.ds(i * 16, 16)] = sidx[pl.ds(i * 16, 16)] + noff
            didx2[pl.ds(i * 16, 16)] = didx[pl.ds(i * 16, 16)] + noff
            return carry
        lax.fori_loop(0, C // 16, _adj, 0)

        cpk = pltpu.async_copy(k_h.at[sidx], kro, semg)
        cpq = pltpu.async_copy(q_h.at[didx2], qro, semg)
        cpe = pltpu.async_copy(e_h.at[pl.ds(cid * NE + ebase, C)], ero, semg)
        cpb = pltpu.async_copy(
            b_h.at[pl.ds(cid * 4, 4), pl.ds(ebase, C)], brot, semg)
        cpk.wait()
        cpq.wait()
        cpe.wait()
        cpb.wait()

        def _p1(g, carry):
            def _edge(el, accs):
                e = g * 16 + el
                p = []
                for sl in range(8):
                    kk = kro[e, pl.ds(sl * 16, 16)]
                    qq = qro[e, pl.ds(sl * 16, 16)]
                    ee = ero[e, pl.ds(sl * 16, 16)]
                    p.append(kk * qq * ee)
                lmask = iota16 == el
                return tuple(
                    jnp.where(lmask, jnp.sum(p[2 * h] + p[2 * h + 1]),
                              accs[h])
                    for h in range(4))
            accs = lax.fori_loop(
                0, 16, _edge, (zero16, zero16, zero16, zero16))
            for h in range(4):
                sbuft[h, pl.ds(g * 16, 16)] = accs[h]
            return carry
        lax.fori_loop(0, C // 16, _p1, 0)

        cpv = pltpu.async_copy(v_h.at[sidx], kro, semv)

        @pl.when(ch > 0)
        def _():
            pltpu.make_async_copy(sbuf, zacc.at[prev_didx], semz).wait()

        for h in range(4):
            for sl in range(C // 16):
                x = sbuft[h, pl.ds(sl * 16, 16)]
                bb = brot[h, pl.ds(sl * 16, 16)]
                x = x * INV_SQRT_DH + bb
                x = jnp.minimum(jnp.maximum(x, -5.0), 5.0)
                x = jnp.exp(x)
                sbuft[h, pl.ds(sl * 16, 16)] = x
                plsc.store_scatter(
                    sbuf, [iota16 + sl * 16, jnp.broadcast_to(h, (16,))], x)

        @pl.when(ch > 0)
        def _():
            pltpu.make_async_copy(msg, wvacc.at[prev_didx], semw).wait()

        cpv.wait()

        def _p2(g, carry):
            rows0 = g * 16
            srows = [sbuft[h, pl.ds(rows0, 16)] for h in range(4)]
            for el in range(16):
                e = rows0 + el
                for h in range(4):
                    s = jnp.broadcast_to(srows[h][el], (16,))
                    msg[e, pl.ds(h * 32, 16)] = kro[e, pl.ds(h * 32, 16)] * s
                    msg[e, pl.ds(h * 32 + 16, 16)] = \
                        kro[e, pl.ds(h * 32 + 16, 16)] * s
            return carry
        lax.fori_loop(0, C // 16, _p2, 0)

        pltpu.async_copy(msg, wvacc.at[didx], semw, add=True)
        pltpu.async_copy(sbuf, zacc.at[didx], semz, add=True)

    def _pair(j, carry):
        _do_chunk(2 * j, didxa, didxb)
        _do_chunk(2 * j + 1, didxb, didxa)
        return carry

    npair = (NCH0 // 2) + jnp.where(sid == NTILES - 1, 2, 0)
    lax.fori_loop(0, npair, _pair, 0)

    pltpu.make_async_copy(msg, wvacc.at[didxb], semw).wait()
    pltpu.make_async_copy(sbuf, zacc.at[didxb], semz).wait()

    plsc.subcore_barrier()
    pltpu.sync_copy(wvacc.at[pl.ds(r0, NPER)],
                    wv_out.at[pl.ds(cid * N + r0, NPER)])
    pltpu.sync_copy(zacc.at[pl.ds(r0, NPER)],
                    z_out.at[pl.ds(cid * N + r0, NPER)])


def _sc_edge_stage(src, dst, q_st, k_st, v_st, e_st, ebias_t):
    mesh = plsc.VectorSubcoreMesh(core_axis_name="c", subcore_axis_name="s",
                                  num_cores=2, num_subcores=NTILES)
    return pl.kernel(
        _sc_body,
        out_type=(jax.ShapeDtypeStruct((2 * N, DHALF), jnp.float32),
                  jax.ShapeDtypeStruct((2 * N, 16), jnp.float32)),
        mesh=mesh,
        compiler_params=pltpu.CompilerParams(use_tc_tiling_on_sc=False,
                                             needs_layout_passes=False),
        scratch_types=[
            pltpu.VMEM((C,), jnp.int32),
            pltpu.VMEM((C,), jnp.int32),
            pltpu.VMEM((C,), jnp.int32),
            pltpu.VMEM((C,), jnp.int32),
            pltpu.VMEM((C, DHALF), jnp.float32),
            pltpu.VMEM((C, DHALF), jnp.float32),
            pltpu.VMEM((C, DHALF), jnp.float32),
            pltpu.VMEM((C, DHALF), jnp.float32),
            pltpu.VMEM((4, C), jnp.float32),
            pltpu.VMEM((C, 16), jnp.float32),
            pltpu.VMEM((4, C), jnp.float32),
            pltpu.VMEM_SHARED((N, DHALF), jnp.float32),
            pltpu.VMEM_SHARED((N, 16), jnp.float32),
            pltpu.SemaphoreType.DMA,
            pltpu.SemaphoreType.DMA,
            pltpu.SemaphoreType.DMA,
            pltpu.SemaphoreType.DMA,
        ],
    )(src, dst, q_st, k_st, v_st, e_st, ebias_t)


_BLK3 = 400
_BN_SCALE = float(1.0 / np.sqrt(1.0 + 1e-5))


def _final_body(h_ref, w0_ref, w1_ref, z0_ref, z1_ref, g1_ref, be1_ref,
                wf1_ref, bf1_ref, wf2_ref, bf2_ref, g2_ref, be2_ref, out_ref):
    hb = h_ref[...]
    lanes = lax.broadcasted_iota(jnp.int32, (4, DHALF), 1)
    ks = lax.broadcasted_iota(jnp.int32, (4, DHALF), 0)
    R = jnp.where(lanes // DH == ks, 1.0, 0.0).astype(jnp.float32)

    def half(w_ref, z_ref):
        wv = w_ref[...]
        z = z_ref[:, 0:4]
        zx = jnp.dot(z, R, preferred_element_type=jnp.float32)
        return wv / (zx + 1e-6)

    attn = jnp.concatenate(
        [half(w0_ref, z0_ref), half(w1_ref, z1_ref)], axis=1)
    h_attn = hb + attn
    h1 = h_attn * (g1_ref[...] * _BN_SCALE) + be1_ref[...]
    ff = jnp.maximum(
        jnp.dot(h1, wf1_ref[...], preferred_element_type=jnp.float32)
        + bf1_ref[...], 0.0)
    ff = jnp.dot(ff, wf2_ref[...], preferred_element_type=jnp.float32) \
        + bf2_ref[...]
    out_ref[...] = (h1 + ff) * (g2_ref[...] * _BN_SCALE) + be2_ref[...]


def _final_stage(h, wv, z, gamma1, beta1, W_ff1, b_ff1, W_ff2, b_ff2,
                 gamma2, beta2):
    nb = N // _BLK3
    grid = (nb,)
    in_specs = [
        pl.BlockSpec((_BLK3, OUT_DIM), lambda i: (i, 0)),
        pl.BlockSpec((_BLK3, DHALF), lambda i: (i, 0)),
        pl.BlockSpec((_BLK3, DHALF), lambda i: (nb + i, 0)),
        pl.BlockSpec((_BLK3, 16), lambda i: (i, 0)),
        pl.BlockSpec((_BLK3, 16), lambda i: (nb + i, 0)),
        pl.BlockSpec((1, OUT_DIM), lambda i: (0, 0)),
        pl.BlockSpec((1, OUT_DIM), lambda i: (0, 0)),
        pl.BlockSpec((OUT_DIM, 2 * OUT_DIM), lambda i: (0, 0)),
        pl.BlockSpec((1, 2 * OUT_DIM), lambda i: (0, 0)),
        pl.BlockSpec((2 * OUT_DIM, OUT_DIM), lambda i: (0, 0)),
        pl.BlockSpec((1, OUT_DIM), lambda i: (0, 0)),
        pl.BlockSpec((1, OUT_DIM), lambda i: (0, 0)),
        pl.BlockSpec((1, OUT_DIM), lambda i: (0, 0)),
    ]
    return pl.pallas_call(
        _final_body,
        grid=grid,
        in_specs=in_specs,
        out_specs=pl.BlockSpec((_BLK3, OUT_DIM), lambda i: (i, 0)),
        out_shape=jax.ShapeDtypeStruct((N, OUT_DIM), jnp.float32),
    )(h, wv, wv, z, z, gamma1.reshape(1, -1), beta1.reshape(1, -1),
      W_ff1, b_ff1.reshape(1, -1), W_ff2, b_ff2.reshape(1, -1),
      gamma2.reshape(1, -1), beta2.reshape(1, -1))


def kernel(h, edge_index, edge_attr, W_Q, W_K, W_V, W_E, W_Eb, b_Eb,
           gamma1, beta1, W_ff1, b_ff1, W_ff2, b_ff2, gamma2, beta2):
    src = edge_index[0]
    dst = edge_index[1]
    q_st, k_st, v_st = _qkv_project(h, W_Q, W_K, W_V)
    e_st, ebias_t = _edge_features(edge_attr, W_E, W_Eb, b_Eb)
    wv, z = _sc_edge_stage(src, dst, q_st, k_st, v_st, e_st, ebias_t)
    h2 = _final_stage(h, wv, z, gamma1, beta1, W_ff1, b_ff1, W_ff2, b_ff2,
                      gamma2, beta2)
    return (h2, edge_attr)


# speedup vs baseline: 20.0176x; 1.0543x over previous
import functools

import jax
import jax.numpy as jnp
import numpy as np
from jax import lax
from jax.experimental import pallas as pl
from jax.experimental.pallas import tpu as pltpu
from jax.experimental.pallas import tpu_sc as plsc

N = 10000
NE = 160000
IN_DIM = 256
OUT_DIM = 256
H = 8
DH = 32
DE = 16

DHALF = 128
NTILES = 16
C = 64
EPT0 = 9984
NCH0 = EPT0 // C
NPER = N // NTILES
INV_SQRT_DH = float(1.0 / np.sqrt(DH))


_BLK1 = 400


def _qkv_body(h_ref, wq_ref, wk_ref, wv_ref, q_out, k_out, v_out):
    hb = h_ref[...]
    q_out[...] = jnp.dot(hb, wq_ref[...], preferred_element_type=jnp.float32)
    k_out[...] = jnp.dot(hb, wk_ref[...], preferred_element_type=jnp.float32)
    v_out[...] = jnp.dot(hb, wv_ref[...], preferred_element_type=jnp.float32)


def _qkv_project(h, W_Q, W_K, W_V):
    nb = N // _BLK1
    grid = (nb, 2)
    in_specs = [
        pl.BlockSpec((_BLK1, IN_DIM), lambda i, j: (i, 0)),
        pl.BlockSpec((IN_DIM, DHALF), lambda i, j: (0, j)),
        pl.BlockSpec((IN_DIM, DHALF), lambda i, j: (0, j)),
        pl.BlockSpec((IN_DIM, DHALF), lambda i, j: (0, j)),
    ]
    out_spec = pl.BlockSpec((_BLK1, DHALF), lambda i, j: (j * nb + i, 0))
    out_sds = jax.ShapeDtypeStruct((2 * N, DHALF), jnp.float32)
    return pl.pallas_call(
        _qkv_body,
        grid=grid,
        in_specs=in_specs,
        out_specs=[out_spec, out_spec, out_spec],
        out_shape=[out_sds, out_sds, out_sds],
    )(h, W_Q, W_K, W_V)


_BLK2 = 1280


def _efeat_body(a_ref, we_ref, webt_ref, bebt_ref, e_out, b_out):
    ab = a_ref[...]
    e_out[...] = jnp.dot(ab, we_ref[...], preferred_element_type=jnp.float32)
    bt = jax.lax.dot_general(
        webt_ref[...], ab, (((1,), (1,)), ((), ())),
        preferred_element_type=jnp.float32)
    b_out[...] = bt + bebt_ref[...]


def _edge_features(edge_attr, W_E, W_Eb, b_Eb):
    nb = NE // _BLK2
    grid = (nb, 2)
    in_specs = [
        pl.BlockSpec((_BLK2, DE), lambda i, j: (i, 0)),
        pl.BlockSpec((DE, DHALF), lambda i, j: (0, j)),
        pl.BlockSpec((H, DE), lambda i, j: (0, 0)),
        pl.BlockSpec((H, 1), lambda i, j: (0, 0)),
    ]
    out_specs = [
        pl.BlockSpec((_BLK2, DHALF), lambda i, j: (j * nb + i, 0)),
        pl.BlockSpec((H, _BLK2), lambda i, j: (0, i)),
    ]
    out_shape = [
        jax.ShapeDtypeStruct((2 * NE, DHALF), jnp.float32),
        jax.ShapeDtypeStruct((H, NE), jnp.float32),
    ]
    return pl.pallas_call(
        _efeat_body,
        grid=grid,
        in_specs=in_specs,
        out_specs=out_specs,
        out_shape=out_shape,
    )(edge_attr, W_E, W_Eb.T, b_Eb.reshape(H, 1))


def _sc_body(src_h, dst_h, q_h, k_h, v_h, e_h, b_h, wv_out, z_out,
             sidxa, sidxb, didxa, didxb, didx2, kro, qro, ero, msg, brot,
             sbuf, sbuft, wvacc, zacc, semg, semv, semw, semz, semi):
    cid = lax.axis_index("c")
    sid = lax.axis_index("s")
    zero16 = jnp.zeros((16,), jnp.float32)
    iota16 = lax.iota(jnp.int32, 16)

    def _zero_m(i, carry):
        msg[i // 8, pl.ds((i % 8) * 16, 16)] = zero16
        return carry
    lax.fori_loop(0, C * 8, _zero_m, 0)

    def _zero_s(i, carry):
        sbuf[i, pl.ds(0, 16)] = zero16
        return carry
    lax.fori_loop(0, C, _zero_s, 0)

    r0 = sid * NPER
    for i in range(9):
        pltpu.sync_copy(msg, wvacc.at[pl.ds(r0 + i * C, C)])
        pltpu.sync_copy(sbuf, zacc.at[pl.ds(r0 + i * C, C)])
    rem = NPER - 9 * C
    pltpu.sync_copy(msg.at[pl.ds(0, rem), :], wvacc.at[pl.ds(r0 + 9 * C, rem)])
    pltpu.sync_copy(sbuf.at[pl.ds(0, rem), :], zacc.at[pl.ds(r0 + 9 * C, rem)])
    plsc.subcore_barrier()

    noff = cid * N
    lane0 = iota16 == 0
    nch = NCH0 + jnp.where(sid == NTILES - 1, 4, 0)

    pltpu.sync_copy(src_h.at[pl.ds(sid * EPT0, C)], sidxa)
    pltpu.sync_copy(dst_h.at[pl.ds(sid * EPT0, C)], didxa)

    def _do_chunk(ch, sidx, didx, sidx_n, didx_n):
        ebase = sid * EPT0 + ch * C
        @pl.when(ch > 0)
        def _():
            pltpu.make_async_copy(
                src_h.at[pl.ds(ebase, C)], sidx, semi).wait()
            pltpu.make_async_copy(
                dst_h.at[pl.ds(ebase, C)], didx, semi).wait()

        def _adj(i, carry):
            sidx[pl.ds(i * 16, 16)] = sidx[pl.ds(i * 16, 16)] + noff
            didx2[pl.ds(i * 16, 16)] = didx[pl.ds(i * 16, 16)] + noff
            return carry
        lax.fori_loop(0, C // 16, _adj, 0)

        cpk = pltpu.async_copy(k_h.at[sidx], kro, semg)
        cpq = pltpu.async_copy(q_h.at[didx2], qro, semg)
        cpe = pltpu.async_copy(e_h.at[pl.ds(cid * NE + ebase, C)], ero, semg)
        cpb = pltpu.async_copy(
            b_h.at[pl.ds(cid * 4, 4), pl.ds(ebase, C)], brot, semg)
        cpk.wait()
        cpq.wait()
        cpe.wait()
        cpb.wait()

        def _p1(g, carry):
            def _edge(el, accs):
                e = g * 16 + el
                p = []
                for sl in range(8):
                    kk = kro[e, pl.ds(sl * 16, 16)]
                    qq = qro[e, pl.ds(sl * 16, 16)]
                    ee = ero[e, pl.ds(sl * 16, 16)]
                    p.append(kk * qq * ee)
                lmask = iota16 == el
                return tuple(
                    jnp.where(lmask, jnp.sum(p[2 * h] + p[2 * h + 1]),
                              accs[h])
                    for h in range(4))
            accs = lax.fori_loop(
                0, 16, _edge, (zero16, zero16, zero16, zero16))
            for h in range(4):
                sbuft[h, pl.ds(g * 16, 16)] = accs[h]
            return carry
        lax.fori_loop(0, C // 16, _p1, 0)

        cpv = pltpu.async_copy(v_h.at[sidx], kro, semv)

        @pl.when(ch > 0)
        def _():
            pltpu.make_async_copy(sbuf, zacc.at[didx_n], semz).wait()

        for h in range(4):
            for sl in range(C // 16):
                x = sbuft[h, pl.ds(sl * 16, 16)]
                bb = brot[h, pl.ds(sl * 16, 16)]
                x = x * INV_SQRT_DH + bb
                x = jnp.minimum(jnp.maximum(x, -5.0), 5.0)
                x = jnp.exp(x)
                sbuft[h, pl.ds(sl * 16, 16)] = x
                plsc.store_scatter(
                    sbuf, [iota16 + sl * 16, jnp.broadcast_to(h, (16,))], x)

        @pl.when(ch > 0)
        def _():
            pltpu.make_async_copy(msg, wvacc.at[didx_n], semw).wait()

        cpv.wait()

        @pl.when(ch + 1 < nch)
        def _():
            pltpu.async_copy(src_h.at[pl.ds(ebase + C, C)], sidx_n, semi)
            pltpu.async_copy(dst_h.at[pl.ds(ebase + C, C)], didx_n, semi)

        def _p2(g, carry):
            rows0 = g * 16
            srows = [sbuft[h, pl.ds(rows0, 16)] for h in range(4)]
            for el in range(16):
                e = rows0 + el
                for h in range(4):
                    s = jnp.broadcast_to(srows[h][el], (16,))
                    msg[e, pl.ds(h * 32, 16)] = kro[e, pl.ds(h * 32, 16)] * s
                    msg[e, pl.ds(h * 32 + 16, 16)] = \
                        kro[e, pl.ds(h * 32 + 16, 16)] * s
            return carry
        lax.fori_loop(0, C // 16, _p2, 0)

        pltpu.async_copy(msg, wvacc.at[didx], semw, add=True)
        pltpu.async_copy(sbuf, zacc.at[didx], semz, add=True)

    def _pair(j, carry):
        _do_chunk(2 * j, sidxa, didxa, sidxb, didxb)
        _do_chunk(2 * j + 1, sidxb, didxb, sidxa, didxa)
        return carry

    npair = (NCH0 // 2) + jnp.where(sid == NTILES - 1, 2, 0)
    lax.fori_loop(0, npair, _pair, 0)

    pltpu.make_async_copy(msg, wvacc.at[didxb], semw).wait()
    pltpu.make_async_copy(sbuf, zacc.at[didxb], semz).wait()

    plsc.subcore_barrier()
    pltpu.sync_copy(wvacc.at[pl.ds(r0, NPER)],
                    wv_out.at[pl.ds(cid * N + r0, NPER)])
    pltpu.sync_copy(zacc.at[pl.ds(r0, NPER)],
                    z_out.at[pl.ds(cid * N + r0, NPER)])


def _sc_edge_stage(src, dst, q_st, k_st, v_st, e_st, ebias_t):
    mesh = plsc.VectorSubcoreMesh(core_axis_name="c", subcore_axis_name="s",
                                  num_cores=2, num_subcores=NTILES)
    return pl.kernel(
        _sc_body,
        out_type=(jax.ShapeDtypeStruct((2 * N, DHALF), jnp.float32),
                  jax.ShapeDtypeStruct((2 * N, 16), jnp.float32)),
        mesh=mesh,
        compiler_params=pltpu.CompilerParams(use_tc_tiling_on_sc=False,
                                             needs_layout_passes=False),
        scratch_types=[
            pltpu.VMEM((C,), jnp.int32),
            pltpu.VMEM((C,), jnp.int32),
            pltpu.VMEM((C,), jnp.int32),
            pltpu.VMEM((C,), jnp.int32),
            pltpu.VMEM((C,), jnp.int32),
            pltpu.VMEM((C, DHALF), jnp.float32),
            pltpu.VMEM((C, DHALF), jnp.float32),
            pltpu.VMEM((C, DHALF), jnp.float32),
            pltpu.VMEM((C, DHALF), jnp.float32),
            pltpu.VMEM((4, C), jnp.float32),
            pltpu.VMEM((C, 16), jnp.float32),
            pltpu.VMEM((4, C), jnp.float32),
            pltpu.VMEM_SHARED((N, DHALF), jnp.float32),
            pltpu.VMEM_SHARED((N, 16), jnp.float32),
            pltpu.SemaphoreType.DMA,
            pltpu.SemaphoreType.DMA,
            pltpu.SemaphoreType.DMA,
            pltpu.SemaphoreType.DMA,
            pltpu.SemaphoreType.DMA,
        ],
    )(src, dst, q_st, k_st, v_st, e_st, ebias_t)


_BLK3 = 400
_BN_SCALE = float(1.0 / np.sqrt(1.0 + 1e-5))


def _final_body(h_ref, w0_ref, w1_ref, z0_ref, z1_ref, g1_ref, be1_ref,
                wf1_ref, bf1_ref, wf2_ref, bf2_ref, g2_ref, be2_ref, out_ref):
    hb = h_ref[...]
    lanes = lax.broadcasted_iota(jnp.int32, (4, DHALF), 1)
    ks = lax.broadcasted_iota(jnp.int32, (4, DHALF), 0)
    R = jnp.where(lanes // DH == ks, 1.0, 0.0).astype(jnp.float32)

    def half(w_ref, z_ref):
        wv = w_ref[...]
        z = z_ref[:, 0:4]
        zx = jnp.dot(z, R, preferred_element_type=jnp.float32)
        return wv / (zx + 1e-6)

    attn = jnp.concatenate(
        [half(w0_ref, z0_ref), half(w1_ref, z1_ref)], axis=1)
    h_attn = hb + attn
    h1 = h_attn * (g1_ref[...] * _BN_SCALE) + be1_ref[...]
    ff = jnp.maximum(
        jnp.dot(h1, wf1_ref[...], preferred_element_type=jnp.float32)
        + bf1_ref[...], 0.0)
    ff = jnp.dot(ff, wf2_ref[...], preferred_element_type=jnp.float32) \
        + bf2_ref[...]
    out_ref[...] = (h1 + ff) * (g2_ref[...] * _BN_SCALE) + be2_ref[...]


def _final_stage(h, wv, z, gamma1, beta1, W_ff1, b_ff1, W_ff2, b_ff2,
                 gamma2, beta2):
    nb = N // _BLK3
    grid = (nb,)
    in_specs = [
        pl.BlockSpec((_BLK3, OUT_DIM), lambda i: (i, 0)),
        pl.BlockSpec((_BLK3, DHALF), lambda i: (i, 0)),
        pl.BlockSpec((_BLK3, DHALF), lambda i: (nb + i, 0)),
        pl.BlockSpec((_BLK3, 16), lambda i: (i, 0)),
        pl.BlockSpec((_BLK3, 16), lambda i: (nb + i, 0)),
        pl.BlockSpec((1, OUT_DIM), lambda i: (0, 0)),
        pl.BlockSpec((1, OUT_DIM), lambda i: (0, 0)),
        pl.BlockSpec((OUT_DIM, 2 * OUT_DIM), lambda i: (0, 0)),
        pl.BlockSpec((1, 2 * OUT_DIM), lambda i: (0, 0)),
        pl.BlockSpec((2 * OUT_DIM, OUT_DIM), lambda i: (0, 0)),
        pl.BlockSpec((1, OUT_DIM), lambda i: (0, 0)),
        pl.BlockSpec((1, OUT_DIM), lambda i: (0, 0)),
        pl.BlockSpec((1, OUT_DIM), lambda i: (0, 0)),
    ]
    return pl.pallas_call(
        _final_body,
        grid=grid,
        in_specs=in_specs,
        out_specs=pl.BlockSpec((_BLK3, OUT_DIM), lambda i: (i, 0)),
        out_shape=jax.ShapeDtypeStruct((N, OUT_DIM), jnp.float32),
    )(h, wv, wv, z, z, gamma1.reshape(1, -1), beta1.reshape(1, -1),
      W_ff1, b_ff1.reshape(1, -1), W_ff2, b_ff2.reshape(1, -1),
      gamma2.reshape(1, -1), beta2.reshape(1, -1))


def kernel(h, edge_index, edge_attr, W_Q, W_K, W_V, W_E, W_Eb, b_Eb,
           gamma1, beta1, W_ff1, b_ff1, W_ff2, b_ff2, gamma2, beta2):
    src = edge_index[0]
    dst = edge_index[1]
    q_st, k_st, v_st = _qkv_project(h, W_Q, W_K, W_V)
    e_st, ebias_t = _edge_features(edge_attr, W_E, W_Eb, b_Eb)
    wv, z = _sc_edge_stage(src, dst, q_st, k_st, v_st, e_st, ebias_t)
    h2 = _final_stage(h, wv, z, gamma1, beta1, W_ff1, b_ff1, W_ff2, b_ff2,
                      gamma2, beta2)
    return (h2, edge_attr)
